# baseline (speedup 1.0000x reference)
import jax
import jax.numpy as jnp
from jax.experimental import pallas as pl
from jax.experimental.pallas import tpu as pltpu

_V = 431
_C = 1280
_S = 16
_G = 16


def _fused_body(h_ref, w1_ref, b1_ref, lnpw_ref, lnpb_ref, l1k_ref, rb1_ref,
                ln1w_ref, ln1b_ref, gk2_ref, bc_ref, adj_ref,
                ln2w_ref, ln2b_ref, l2k_ref, rb2_ref, w3_ref, b3_ref,
                out_ref, a2t_scr, w1t_scr, w3t_scr):
    @pl.when(pl.program_id(0) == 0)
    def _():
        w1t_scr[...] = jnp.transpose(w1_ref[...], (1, 0))
        w3t_scr[...] = jnp.transpose(w3_ref[...], (1, 0))
        adjt = jnp.transpose(adj_ref[...], (1, 0))
        a2t_scr[...] = jnp.dot(adjt, adjt, preferred_element_type=jnp.float32)

    def ln_relu(v, f, w_ref, b_ref):
        v3 = v.reshape(_G, f, _V)
        u = jnp.mean(v3, axis=1, keepdims=True)
        s2 = jnp.mean((v3 - u) ** 2, axis=1, keepdims=True)
        t = (w_ref[...][None, :, :] * (v3 - u) * jax.lax.rsqrt(s2 + 1e-12)
             + b_ref[...][None, :, :])
        return jnp.maximum(t, 0.0).reshape(_G * f, _V)

    x1 = jnp.dot(h_ref[...], w1t_scr[...],
                 preferred_element_type=jnp.float32) + b1_ref[...]
    t = ln_relu(x1, _S, lnpw_ref, lnpb_ref)
    y = jnp.dot(l1k_ref[...], t, preferred_element_type=jnp.float32) + rb1_ref[...]
    y = ln_relu(y, 8, ln1w_ref, ln1b_ref)
    q = jnp.dot(gk2_ref[...], y, preferred_element_type=jnp.float32)
    y = jnp.dot(q, a2t_scr[...], preferred_element_type=jnp.float32) + bc_ref[...]
    t2 = ln_relu(y, 8, ln2w_ref, ln2b_ref)
    y2 = jnp.dot(l2k_ref[...], t2, preferred_element_type=jnp.float32) + rb2_ref[...]
    z = x1 + y2
    out_ref[...] = jnp.dot(z, w3t_scr[...],
                           preferred_element_type=jnp.float32) + b3_ref[...]


def kernel(hidden_states, W1, b1, ln_pre_w, ln_pre_b, lin1_w, lin1_b,
           ln1_w, ln1_b, gcn_w, gcn_b, adjmat, ln2_w, ln2_b,
           lin2_w, lin2_b, W3, b3):
    T = hidden_states.shape[2]
    B = hidden_states.shape[0]
    h5 = hidden_states.reshape(B, T, 40, T, _S)
    one = 1.0 + 0.0 * W1[0, 0]
    ht = (jnp.transpose(h5, (0, 1, 4, 2, 3)) * one).reshape(B * T * _S, _C)
    n = B * T
    ng = n // _G

    eye = jnp.eye(_G, dtype=jnp.float32)
    l1k = jnp.kron(eye, lin1_w)
    gk2 = jnp.kron(eye, (gcn_w @ gcn_w).T)
    l2k = jnp.kron(eye, lin2_w)
    rb1 = jnp.tile(lin1_b, _G)[:, None]
    rb2 = jnp.tile(lin2_b, _G)[:, None]
    r = adjmat.sum(axis=1)
    bct = (gcn_b @ gcn_w)[:, None] * r[None, :] + gcn_b[:, None]
    bc = jnp.tile(bct, (_G, 1))

    const = lambda i: (0, 0)
    out = pl.pallas_call(
        _fused_body,
        grid=(ng,),
        in_specs=[
            pl.BlockSpec((_G * _S, _C), lambda i: (i, 0)),
            pl.BlockSpec((_V, _C), const),
            pl.BlockSpec((1, _V), const),
            pl.BlockSpec((_S, 1), const),
            pl.BlockSpec((_S, 1), const),
            pl.BlockSpec((_G * 8, _G * _S), const),
            pl.BlockSpec((_G * 8, 1), const),
            pl.BlockSpec((8, 1), const),
            pl.BlockSpec((8, 1), const),
            pl.BlockSpec((_G * 8, _G * 8), const),
            pl.BlockSpec((_G * 8, _V), const),
            pl.BlockSpec((_V, _V), const),
            pl.BlockSpec((8, 1), const),
            pl.BlockSpec((8, 1), const),
            pl.BlockSpec((_G * _S, _G * 8), const),
            pl.BlockSpec((_G * _S, 1), const),
            pl.BlockSpec((_C, _V), const),
            pl.BlockSpec((1, _C), const),
        ],
        out_specs=pl.BlockSpec((_G * _S, _C), lambda i: (i, 0)),
        out_shape=jax.ShapeDtypeStruct((n * _S, _C), jnp.float32),
        scratch_shapes=[pltpu.VMEM((_V, _V), jnp.float32),
                        pltpu.VMEM((_C, _V), jnp.float32),
                        pltpu.VMEM((_V, _C), jnp.float32)],
    )(ht, W1, b1[None, :], ln_pre_w[:, None], ln_pre_b[:, None],
      l1k, rb1, ln1_w[:, None], ln1_b[:, None], gk2, bc, adjmat,
      ln2_w[:, None], ln2_b[:, None], l2k, rb2, W3, b3[None, :])

    o5 = out.reshape(B, T, _S, 40, T)
    return (jnp.transpose(o5, (0, 1, 3, 4, 2)) * one).reshape(B, _C, T, 4, 4)


# speedup vs baseline: 1.3216x; 1.3216x over previous
import jax
import jax.numpy as jnp
from jax.experimental import pallas as pl
from jax.experimental.pallas import tpu as pltpu

_V = 431
_C = 1280
_S = 16
_G = 16
_T = 32
_C2 = _C // _T


def _to_rows(h):
    a = h.reshape(_G, _C2, _T * _S)
    a = jnp.transpose(a, (0, 2, 1))
    a = a.reshape(_G, _T, _S, _C2)
    a = jnp.transpose(a, (0, 1, 3, 2))
    a = a.reshape(_G, _C, _S)
    a = jnp.transpose(a, (0, 2, 1))
    return a.reshape(_G * _S, _C)


def _from_rows(o):
    b = o.reshape(_G, _S, _C)
    b = jnp.transpose(b, (0, 2, 1))
    b = b.reshape(_G, _T, _C2, _S)
    b = jnp.transpose(b, (0, 1, 3, 2))
    b = b.reshape(_G, _T * _S, _C2)
    b = jnp.transpose(b, (0, 2, 1))
    return b.reshape(_G * _C2, _T * _S)


def _fused_body(h_ref, w1_ref, b1_ref, lnpw_ref, lnpb_ref, l1k_ref, rb1_ref,
                ln1w_ref, ln1b_ref, gk2_ref, bc_ref, adj_ref,
                ln2w_ref, ln2b_ref, l2k_ref, rb2_ref, w3_ref, b3p_ref,
                out_ref, a2t_scr, w1p_scr, w3p_scr):
    @pl.when(pl.program_id(0) == 0)
    def _():
        i0 = jax.lax.broadcasted_iota(jnp.int32, (_C, _C), 0)
        i1 = jax.lax.broadcasted_iota(jnp.int32, (_C, _C), 1)
        perm = ((i0 % _C2) * _T + i0 // _C2 == i1).astype(jnp.float32)
        w1p_scr[...] = jnp.dot(perm, jnp.transpose(w1_ref[...], (1, 0)),
                               preferred_element_type=jnp.float32)
        w3p_scr[...] = jnp.transpose(
            jnp.dot(perm, w3_ref[...], preferred_element_type=jnp.float32),
            (1, 0))
        adjt = jnp.transpose(adj_ref[...], (1, 0))
        a2t_scr[...] = jnp.dot(adjt, adjt, preferred_element_type=jnp.float32)

    def ln_relu(v, f, w_ref, b_ref):
        v3 = v.reshape(_G, f, _V)
        u = jnp.mean(v3, axis=1, keepdims=True)
        s2 = jnp.mean((v3 - u) ** 2, axis=1, keepdims=True)
        t = (w_ref[...][None, :, :] * (v3 - u) * jax.lax.rsqrt(s2 + 1e-12)
             + b_ref[...][None, :, :])
        return jnp.maximum(t, 0.0).reshape(_G * f, _V)

    ht = _to_rows(h_ref[...])
    x1 = jnp.dot(ht, w1p_scr[...],
                 preferred_element_type=jnp.float32) + b1_ref[...]
    t = ln_relu(x1, _S, lnpw_ref, lnpb_ref)
    y = jnp.dot(l1k_ref[...], t, preferred_element_type=jnp.float32) + rb1_ref[...]
    y = ln_relu(y, 8, ln1w_ref, ln1b_ref)
    q = jnp.dot(gk2_ref[...], y, preferred_element_type=jnp.float32)
    y = jnp.dot(q, a2t_scr[...], preferred_element_type=jnp.float32) + bc_ref[...]
    t2 = ln_relu(y, 8, ln2w_ref, ln2b_ref)
    y2 = jnp.dot(l2k_ref[...], t2, preferred_element_type=jnp.float32) + rb2_ref[...]
    z = x1 + y2
    o = jnp.dot(z, w3p_scr[...],
                preferred_element_type=jnp.float32) + b3p_ref[...]
    out_ref[...] = _from_rows(o)


def kernel(hidden_states, W1, b1, ln_pre_w, ln_pre_b, lin1_w, lin1_b,
           ln1_w, ln1_b, gcn_w, gcn_b, adjmat, ln2_w, ln2_b,
           lin2_w, lin2_b, W3, b3):
    T = hidden_states.shape[2]
    B = hidden_states.shape[0]
    hs2 = hidden_states.reshape(B * _C, T * 16)
    n = B * T
    ng = n // _G

    eye = jnp.eye(_G, dtype=jnp.float32)
    l1k = jnp.kron(eye, lin1_w)
    gk2 = jnp.kron(eye, (gcn_w @ gcn_w).T)
    l2k = jnp.kron(eye, lin2_w)
    rb1 = jnp.tile(lin1_b, _G)[:, None]
    rb2 = jnp.tile(lin2_b, _G)[:, None]
    r = adjmat.sum(axis=1)
    bct = (gcn_b @ gcn_w)[:, None] * r[None, :] + gcn_b[:, None]
    bc = jnp.tile(bct, (_G, 1))
    b3p = b3.reshape(_C2, _T).T.reshape(1, _C)

    const = lambda i: (0, 0)
    out = pl.pallas_call(
        _fused_body,
        grid=(ng,),
        in_specs=[
            pl.BlockSpec((_G * _C2, _T * _S), lambda i: (i, 0)),
            pl.BlockSpec((_V, _C), const),
            pl.BlockSpec((1, _V), const),
            pl.BlockSpec((_S, 1), const),
            pl.BlockSpec((_S, 1), const),
            pl.BlockSpec((_G * 8, _G * _S), const),
            pl.BlockSpec((_G * 8, 1), const),
            pl.BlockSpec((8, 1), const),
            pl.BlockSpec((8, 1), const),
            pl.BlockSpec((_G * 8, _G * 8), const),
            pl.BlockSpec((_G * 8, _V), const),
            pl.BlockSpec((_V, _V), const),
            pl.BlockSpec((8, 1), const),
            pl.BlockSpec((8, 1), const),
            pl.BlockSpec((_G * _S, _G * 8), const),
            pl.BlockSpec((_G * _S, 1), const),
            pl.BlockSpec((_C, _V), const),
            pl.BlockSpec((1, _C), const),
        ],
        out_specs=pl.BlockSpec((_G * _C2, _T * _S), lambda i: (i, 0)),
        out_shape=jax.ShapeDtypeStruct((B * _C, T * 16), jnp.float32),
        scratch_shapes=[pltpu.VMEM((_V, _V), jnp.float32),
                        pltpu.VMEM((_C, _V), jnp.float32),
                        pltpu.VMEM((_V, _C), jnp.float32)],
    )(hs2, W1, b1[None, :], ln_pre_w[:, None], ln_pre_b[:, None],
      l1k, rb1, ln1_w[:, None], ln1_b[:, None], gk2, bc, adjmat,
      ln2_w[:, None], ln2_b[:, None], l2k, rb2, W3, b3p)

    return out.reshape(B, _C, T, 4, 4)


# speedup vs baseline: 1.3446x; 1.0174x over previous
import jax
import jax.numpy as jnp
from jax.experimental import pallas as pl
from jax.experimental.pallas import tpu as pltpu

_V = 431
_C = 1280
_S = 16
_G = 16
_T = 32
_C2 = _C // _T


def _to_rows(h):
    a = h.reshape(_G, _C2, _T * _S)
    a = jnp.transpose(a, (0, 2, 1))
    a = a.reshape(_G, _T, _S, _C2)
    a = jnp.transpose(a, (0, 1, 3, 2))
    a = a.reshape(_G, _C, _S)
    a = jnp.transpose(a, (0, 2, 1))
    return a.reshape(_G * _S, _C)


def _from_rows(o):
    b = o.reshape(_G, _S, _C)
    b = jnp.transpose(b, (0, 2, 1))
    b = b.reshape(_G, _T, _C2, _S)
    b = jnp.transpose(b, (0, 1, 3, 2))
    b = b.reshape(_G, _T * _S, _C2)
    b = jnp.transpose(b, (0, 2, 1))
    return b.reshape(_G * _C2, _T * _S)


def _iota2(shape, dim):
    return jax.lax.broadcasted_iota(jnp.int32, shape, dim)


def _fused_body(h_ref, w1_ref, b1_ref, lnpw_ref, lnpb_ref, l1w_ref, l1b_ref,
                ln1w_ref, ln1b_ref, gw_ref, gb_ref, adj_ref,
                ln2w_ref, ln2b_ref, l2w_ref, l2b_ref, w3_ref, b3_ref,
                out_ref, a2t_scr, w1p_scr, w3p_scr,
                l1k_scr, gk2_scr, l2k_scr, bc_scr, rb1_scr, rb2_scr, b3p_scr):
    @pl.when(pl.program_id(0) == 0)
    def _():
        f32 = jnp.float32
        i0 = _iota2((_C, _C), 0)
        i1 = _iota2((_C, _C), 1)
        perm = ((i0 % _C2) * _T + i0 // _C2 == i1).astype(f32)
        w1p_scr[...] = jnp.dot(perm, jnp.transpose(w1_ref[...], (1, 0)),
                               preferred_element_type=f32)
        w3p_scr[...] = jnp.transpose(
            jnp.dot(perm, w3_ref[...], preferred_element_type=f32), (1, 0))
        b3p_scr[...] = jnp.transpose(
            jnp.dot(perm, b3_ref[...], preferred_element_type=f32), (1, 0))
        adjt = jnp.transpose(adj_ref[...], (1, 0))
        a2t_scr[...] = jnp.dot(adjt, adjt, preferred_element_type=f32)
        st8 = (_iota2((_G * 8, 8), 0) % 8 == _iota2((_G * 8, 8), 1)).astype(f32)
        st16 = (_iota2((_G * _S, _S), 0) % _S
                == _iota2((_G * _S, _S), 1)).astype(f32)
        t16 = (_iota2((_S, _G * _S), 0)
               == _iota2((_S, _G * _S), 1) % _S).astype(f32)
        t8 = (_iota2((8, _G * 8), 0) == _iota2((8, _G * 8), 1) % 8).astype(f32)
        mk1 = (_iota2((_G * 8, _G * _S), 0) // 8
               == _iota2((_G * 8, _G * _S), 1) // _S).astype(f32)
        mkg = (_iota2((_G * 8, _G * 8), 0) // 8
               == _iota2((_G * 8, _G * 8), 1) // 8).astype(f32)
        mk2 = (_iota2((_G * _S, _G * 8), 0) // _S
               == _iota2((_G * _S, _G * 8), 1) // 8).astype(f32)
        l1k_scr[...] = jnp.dot(jnp.dot(st8, l1w_ref[...],
                                       preferred_element_type=f32), t16,
                               preferred_element_type=f32) * mk1
        gw2t = jnp.transpose(jnp.dot(gw_ref[...], gw_ref[...],
                                     preferred_element_type=f32), (1, 0))
        gk2_scr[...] = jnp.dot(jnp.dot(st8, gw2t,
                                       preferred_element_type=f32), t8,
                               preferred_element_type=f32) * mkg
        l2k_scr[...] = jnp.dot(jnp.dot(st16, l2w_ref[...],
                                       preferred_element_type=f32), t8,
                               preferred_element_type=f32) * mk2
        rb1_scr[...] = jnp.dot(st8, l1b_ref[...], preferred_element_type=f32)
        rb2_scr[...] = jnp.dot(st16, l2b_ref[...], preferred_element_type=f32)
        r_row = jnp.sum(adjt, axis=0, keepdims=True)
        bw = jax.lax.dot_general(gw_ref[...], gb_ref[...],
                                 (((0,), (0,)), ((), ())),
                                 preferred_element_type=f32)
        bct = bw * r_row + gb_ref[...]
        bc_scr[...] = jnp.dot(st8, bct, preferred_element_type=f32)

    def ln_relu(v, f, w_ref, b_ref):
        v3 = v.reshape(_G, f, _V)
        u = jnp.mean(v3, axis=1, keepdims=True)
        s2 = jnp.mean((v3 - u) ** 2, axis=1, keepdims=True)
        t = (w_ref[...][None, :, :] * (v3 - u) * jax.lax.rsqrt(s2 + 1e-12)
             + b_ref[...][None, :, :])
        return jnp.maximum(t, 0.0).reshape(_G * f, _V)

    ht = _to_rows(h_ref[...])
    x1 = jnp.dot(ht, w1p_scr[...],
                 preferred_element_type=jnp.float32) + b1_ref[...]
    t = ln_relu(x1, _S, lnpw_ref, lnpb_ref)
    y = jnp.dot(l1k_scr[...], t, preferred_element_type=jnp.float32) + rb1_scr[...]
    y = ln_relu(y, 8, ln1w_ref, ln1b_ref)
    q = jnp.dot(gk2_scr[...], y, preferred_element_type=jnp.float32)
    y = jnp.dot(q, a2t_scr[...], preferred_element_type=jnp.float32) + bc_scr[...]
    t2 = ln_relu(y, 8, ln2w_ref, ln2b_ref)
    y2 = jnp.dot(l2k_scr[...], t2, preferred_element_type=jnp.float32) + rb2_scr[...]
    z = x1 + y2
    o = jnp.dot(z, w3p_scr[...],
                preferred_element_type=jnp.float32) + b3p_scr[...]
    out_ref[...] = _from_rows(o)


def kernel(hidden_states, W1, b1, ln_pre_w, ln_pre_b, lin1_w, lin1_b,
           ln1_w, ln1_b, gcn_w, gcn_b, adjmat, ln2_w, ln2_b,
           lin2_w, lin2_b, W3, b3):
    T = hidden_states.shape[2]
    B = hidden_states.shape[0]
    hs2 = hidden_states.reshape(B * _C, T * 16)
    n = B * T
    ng = n // _G

    const = lambda i: (0, 0)
    out = pl.pallas_call(
        _fused_body,
        grid=(ng,),
        in_specs=[
            pl.BlockSpec((_G * _C2, _T * _S), lambda i: (i, 0)),
            pl.BlockSpec((_V, _C), const),
            pl.BlockSpec((1, _V), const),
            pl.BlockSpec((_S, 1), const),
            pl.BlockSpec((_S, 1), const),
            pl.BlockSpec((8, _S), const),
            pl.BlockSpec((8, 1), const),
            pl.BlockSpec((8, 1), const),
            pl.BlockSpec((8, 1), const),
            pl.BlockSpec((8, 8), const),
            pl.BlockSpec((8, 1), const),
            pl.BlockSpec((_V, _V), const),
            pl.BlockSpec((8, 1), const),
            pl.BlockSpec((8, 1), const),
            pl.BlockSpec((_S, 8), const),
            pl.BlockSpec((_S, 1), const),
            pl.BlockSpec((_C, _V), const),
            pl.BlockSpec((_C, 1), const),
        ],
        out_specs=pl.BlockSpec((_G * _C2, _T * _S), lambda i: (i, 0)),
        out_shape=jax.ShapeDtypeStruct((B * _C, T * 16), jnp.float32),
        scratch_shapes=[pltpu.VMEM((_V, _V), jnp.float32),
                        pltpu.VMEM((_C, _V), jnp.float32),
                        pltpu.VMEM((_V, _C), jnp.float32),
                        pltpu.VMEM((_G * 8, _G * _S), jnp.float32),
                        pltpu.VMEM((_G * 8, _G * 8), jnp.float32),
                        pltpu.VMEM((_G * _S, _G * 8), jnp.float32),
                        pltpu.VMEM((_G * 8, _V), jnp.float32),
                        pltpu.VMEM((_G * 8, 1), jnp.float32),
                        pltpu.VMEM((_G * _S, 1), jnp.float32),
                        pltpu.VMEM((1, _C), jnp.float32)],
    )(hs2, W1, b1[None, :], ln_pre_w[:, None], ln_pre_b[:, None],
      lin1_w, lin1_b[:, None], ln1_w[:, None], ln1_b[:, None],
      gcn_w, gcn_b[:, None], adjmat,
      ln2_w[:, None], ln2_b[:, None], lin2_w, lin2_b[:, None],
      W3, b3[:, None])

    return out.reshape(B, _C, T, 4, 4)
